# Initial kernel scaffold; baseline (speedup 1.0000x reference)
import jax
import jax.numpy as jnp
from jax.experimental import pallas as pl


def kernel(x, adj_t, edge_weight, W1, b1, W2, b2):
    raise NotImplementedError("write your pallas kernel here")


# speedup vs baseline: 12.8657x; 12.8657x over previous
import functools

import jax
import jax.numpy as jnp
from jax import lax
from jax.experimental import pallas as pl
from jax.experimental.pallas import tpu as pltpu
from jax.experimental.pallas import tpu_sc as plsc

N = 10000
E = 320000
NC, NS, L = 2, 16, 16
NW = NC * NS
CHUNK = 128
CH = 79
EPW = CH * CHUNK
EPAD = NW * EPW
NPAD = 10240
RPW = NPAD // NS
D1 = 128
D2 = 48
BR = 1000
GRID = N // BR


def _mesh():
    return plsc.VectorSubcoreMesh(
        core_axis_name="c", subcore_axis_name="s", num_cores=NC, num_subcores=NS
    )


_SC_PARAMS = pltpu.CompilerParams(use_tc_tiling_on_sc=False)


def _deg_body(dst_hbm, w1_hbm, w2_hbm, out_hbm, dst_v, w1_v, w2_v, zb, acc1, acc2):
    c = lax.axis_index("c")
    s = lax.axis_index("s")
    wid = c * NS + s

    def zb_init(i, _):
        zb[pl.ds(i * L, L)] = jnp.zeros((L,), jnp.float32)
        return 0

    lax.fori_loop(0, RPW // L, zb_init, 0)
    pltpu.sync_copy(zb, acc1.at[pl.ds(s * RPW, RPW)])
    pltpu.sync_copy(zb, acc2.at[pl.ds(s * RPW, RPW)])
    plsc.subcore_barrier()

    pltpu.sync_copy(dst_hbm.at[wid], dst_v)
    pltpu.sync_copy(w1_hbm.at[wid], w1_v)
    pltpu.sync_copy(w2_hbm.at[wid], w2_v)

    def step(j, _):
        pltpu.sync_copy(w1_v.at[j], acc1.at[dst_v.at[j]], add=True)
        pltpu.sync_copy(w2_v.at[j], acc2.at[dst_v.at[j]], add=True)
        return 0

    lax.fori_loop(0, CH, step, 0)
    plsc.subcore_barrier()
    pltpu.sync_copy(acc1.at[pl.ds(s * RPW, RPW)], out_hbm.at[c, 0, pl.ds(s * RPW, RPW)])
    pltpu.sync_copy(acc2.at[pl.ds(s * RPW, RPW)], out_hbm.at[c, 1, pl.ds(s * RPW, RPW)])


def _deg_call(dstp, w1p, w2p):
    f = pl.kernel(
        _deg_body,
        out_type=jax.ShapeDtypeStruct((NC, 2, NPAD), jnp.float32),
        mesh=_mesh(),
        compiler_params=_SC_PARAMS,
        scratch_types=[
            pltpu.VMEM((CH, CHUNK), jnp.int32),
            pltpu.VMEM((CH, CHUNK), jnp.float32),
            pltpu.VMEM((CH, CHUNK), jnp.float32),
            pltpu.VMEM((RPW,), jnp.float32),
            pltpu.VMEM_SHARED((NPAD,), jnp.float32),
            pltpu.VMEM_SHARED((NPAD,), jnp.float32),
        ],
    )
    return f(dstp, w1p, w2p)


DH = 64


def _zero_acc(buf, acc, s, D):
    nq = D // L

    def zrow(r, _):
        for q in range(nq):
            buf[r, pl.ds(q * L, L)] = jnp.zeros((L,), jnp.float32)
        return 0

    lax.fori_loop(0, CHUNK, zrow, 0)
    for k in range(RPW // CHUNK):
        pltpu.sync_copy(buf, acc.at[pl.ds(s * RPW + k * CHUNK, CHUNK)])


def _edge1_body(ha_hbm, hb_hbm, src_hbm, dst_hbm, w_hbm, out_hbm,
                src_v, dst_v, w_v, buf, acc, sem):
    c = lax.axis_index("c")
    s = lax.axis_index("s")
    wid = c * NS + s

    pltpu.sync_copy(src_hbm.at[wid], src_v)
    pltpu.sync_copy(dst_hbm.at[wid], dst_v)

    for phase, h_hbm in enumerate((ha_hbm, hb_hbm)):
        _zero_acc(buf, acc, s, DH)
        plsc.subcore_barrier()

        def step(j, _):
            pltpu.async_copy(h_hbm.at[src_v.at[j]], buf, sem).wait()
            pltpu.sync_copy(w_hbm.at[wid, j], w_v)

            def srow(r, _):
                w = w_v[r]
                for q in range(DH // L):
                    buf[r, pl.ds(q * L, L)] = buf[r, pl.ds(q * L, L)] * w
                return 0

            lax.fori_loop(0, CHUNK, srow, 0)
            pltpu.sync_copy(buf, acc.at[dst_v.at[j]], add=True)
            return 0

        lax.fori_loop(0, CH, step, 0)
        plsc.subcore_barrier()
        pltpu.sync_copy(
            acc.at[pl.ds(s * RPW, RPW)], out_hbm.at[c, phase, pl.ds(s * RPW, RPW)]
        )


def _edge1_call(ha, hb, srcp, dstp, wp):
    f = pl.kernel(
        _edge1_body,
        out_type=jax.ShapeDtypeStruct((NC, 2, NPAD, DH), jnp.float32),
        mesh=_mesh(),
        compiler_params=_SC_PARAMS,
        scratch_types=[
            pltpu.VMEM((CH, CHUNK), jnp.int32),
            pltpu.VMEM((CH, CHUNK), jnp.int32),
            pltpu.VMEM((CHUNK, L), jnp.float32),
            pltpu.VMEM((CHUNK, DH), jnp.float32),
            pltpu.VMEM_SHARED((NPAD, DH), jnp.float32),
            pltpu.SemaphoreType.DMA,
        ],
    )
    wrep = jnp.broadcast_to(wp[..., None], (NW, CH, CHUNK, L))
    return f(ha, hb, srcp, dstp, wrep)


def _edge2_body(h_hbm, src_hbm, dst_hbm, out_hbm, src_v, dst_v, buf, acc, sem):
    c = lax.axis_index("c")
    s = lax.axis_index("s")
    wid = c * NS + s

    _zero_acc(buf, acc, s, D2)
    plsc.subcore_barrier()

    pltpu.sync_copy(src_hbm.at[wid], src_v)
    pltpu.sync_copy(dst_hbm.at[wid], dst_v)

    def step(j, _):
        pltpu.async_copy(h_hbm.at[src_v.at[j]], buf, sem).wait()
        pltpu.sync_copy(buf, acc.at[dst_v.at[j]], add=True)
        return 0

    lax.fori_loop(0, CH, step, 0)
    plsc.subcore_barrier()
    pltpu.sync_copy(acc.at[pl.ds(s * RPW, RPW)], out_hbm.at[c, pl.ds(s * RPW, RPW)])


def _edge2_call(h, srcp, dstp):
    f = pl.kernel(
        _edge2_body,
        out_type=jax.ShapeDtypeStruct((NC, NPAD, D2), jnp.float32),
        mesh=_mesh(),
        compiler_params=_SC_PARAMS,
        scratch_types=[
            pltpu.VMEM((CH, CHUNK), jnp.int32),
            pltpu.VMEM((CH, CHUNK), jnp.int32),
            pltpu.VMEM((CHUNK, D2), jnp.float32),
            pltpu.VMEM_SHARED((NPAD, D2), jnp.float32),
            pltpu.SemaphoreType.DMA,
        ],
    )
    return f(h, srcp, dstp)


def _prep_body(x_ref, w1_ref, d1_ref, d2_ref, ha_ref, hb_ref, i1_ref, i2_ref):
    deg1 = d1_ref[0] + d1_ref[1] + 1.0
    deg2 = d2_ref[0] + d2_ref[1] + 1.0
    dinv1 = jnp.where(deg1 > 0, lax.rsqrt(deg1), 0.0)
    dinv2 = jnp.where(deg2 > 0, lax.rsqrt(deg2), 0.0)
    h1 = jnp.dot(x_ref[...], w1_ref[...], preferred_element_type=jnp.float32)
    h1p = h1 * dinv1
    ha_ref[...] = h1p[:, :DH]
    hb_ref[...] = h1p[:, DH:]
    i1_ref[...] = dinv1
    i2_ref[...] = dinv2


def _prep_call(x, W1, d1, d2):
    return pl.pallas_call(
        _prep_body,
        grid=(GRID,),
        in_specs=[
            pl.BlockSpec((BR, D1), lambda i: (i, 0)),
            pl.BlockSpec((D1, D1), lambda i: (0, 0)),
            pl.BlockSpec((NC, BR, 1), lambda i: (0, i, 0)),
            pl.BlockSpec((NC, BR, 1), lambda i: (0, i, 0)),
        ],
        out_specs=[
            pl.BlockSpec((BR, DH), lambda i: (i, 0)),
            pl.BlockSpec((BR, DH), lambda i: (i, 0)),
            pl.BlockSpec((BR, 1), lambda i: (i, 0)),
            pl.BlockSpec((BR, 1), lambda i: (i, 0)),
        ],
        out_shape=[
            jax.ShapeDtypeStruct((N, DH), jnp.float32),
            jax.ShapeDtypeStruct((N, DH), jnp.float32),
            jax.ShapeDtypeStruct((N, 1), jnp.float32),
            jax.ShapeDtypeStruct((N, 1), jnp.float32),
        ],
    )(x, W1, d1, d2)


def _mid_body(p_ref, ha_ref, hb_ref, i1_ref, i2_ref, b1_ref, w2_ref, h2p_ref):
    sa = p_ref[0, 0] + p_ref[1, 0] + ha_ref[...]
    sb = p_ref[0, 1] + p_ref[1, 1] + hb_ref[...]
    ssum = jnp.concatenate([sa, sb], axis=1)
    y = jnp.maximum(ssum * i1_ref[...] + b1_ref[...], 0.0)
    h2 = jnp.dot(y, w2_ref[...], preferred_element_type=jnp.float32)
    h2p_ref[...] = h2 * i2_ref[...]


def _mid_call(p1, ha, hb, i1, i2, b1r, W2pad):
    return pl.pallas_call(
        _mid_body,
        grid=(GRID,),
        in_specs=[
            pl.BlockSpec((NC, 2, BR, DH), lambda i: (0, 0, i, 0)),
            pl.BlockSpec((BR, DH), lambda i: (i, 0)),
            pl.BlockSpec((BR, DH), lambda i: (i, 0)),
            pl.BlockSpec((BR, 1), lambda i: (i, 0)),
            pl.BlockSpec((BR, 1), lambda i: (i, 0)),
            pl.BlockSpec((1, D1), lambda i: (0, 0)),
            pl.BlockSpec((D1, D2), lambda i: (0, 0)),
        ],
        out_specs=pl.BlockSpec((BR, D2), lambda i: (i, 0)),
        out_shape=jax.ShapeDtypeStruct((N, D2), jnp.float32),
    )(p1, ha, hb, i1, i2, b1r, W2pad)


def _out_body(p_ref, h2p_ref, i2_ref, b2_ref, o_ref):
    z = (p_ref[0] + p_ref[1] + h2p_ref[...]) * i2_ref[...] + b2_ref[...]
    mask = lax.broadcasted_iota(jnp.int32, z.shape, 1) < 40
    zm = jnp.where(mask, z, -jnp.inf)
    m = jnp.max(zm, axis=1, keepdims=True)
    e = jnp.where(mask, jnp.exp(z - m), 0.0)
    o_ref[...] = z - m - jnp.log(jnp.sum(e, axis=1, keepdims=True))


def _out_call(p2, h2p, i2, b2r):
    return pl.pallas_call(
        _out_body,
        grid=(GRID,),
        in_specs=[
            pl.BlockSpec((NC, BR, D2), lambda i: (0, i, 0)),
            pl.BlockSpec((BR, D2), lambda i: (i, 0)),
            pl.BlockSpec((BR, 1), lambda i: (i, 0)),
            pl.BlockSpec((1, D2), lambda i: (0, 0)),
        ],
        out_specs=pl.BlockSpec((BR, D2), lambda i: (i, 0)),
        out_shape=jax.ShapeDtypeStruct((N, D2), jnp.float32),
    )(p2, h2p, i2, b2r)


def kernel(x, adj_t, edge_weight, W1, b1, W2, b2):
    x = x.astype(jnp.float32)
    src = adj_t[0].astype(jnp.int32)
    dst = adj_t[1].astype(jnp.int32)
    pad = EPAD - E
    srcp = jnp.concatenate([src, jnp.zeros((pad,), jnp.int32)]).reshape(NW, CH, CHUNK)
    dstp = jnp.concatenate([dst, jnp.full((pad,), N, jnp.int32)]).reshape(NW, CH, CHUNK)
    w1p = jnp.concatenate(
        [edge_weight.astype(jnp.float32), jnp.zeros((pad,), jnp.float32)]
    ).reshape(NW, CH, CHUNK)
    w2p = jnp.concatenate(
        [jnp.ones((E,), jnp.float32), jnp.zeros((pad,), jnp.float32)]
    ).reshape(NW, CH, CHUNK)
    W2pad = jnp.pad(W2, ((0, 0), (0, D2 - 40)))
    b2r = jnp.pad(b2, (0, D2 - 40)).reshape(1, D2)
    b1r = b1.reshape(1, D1)

    degp = _deg_call(dstp, w1p, w2p)
    d1 = degp[:, 0, :N].reshape(NC, N, 1)
    d2 = degp[:, 1, :N].reshape(NC, N, 1)
    ha, hb, i1, i2 = _prep_call(x, W1, d1, d2)
    p1 = _edge1_call(ha, hb, srcp, dstp, w1p)
    h2p = _mid_call(p1, ha, hb, i1, i2, b1r, W2pad)
    p2 = _edge2_call(h2p, srcp, dstp)
    o = _out_call(p2, h2p, i2, b2r)
    return o[:, :40]


# speedup vs baseline: 13.9081x; 1.0810x over previous
import functools

import jax
import jax.numpy as jnp
from jax import lax
from jax.experimental import pallas as pl
from jax.experimental.pallas import tpu as pltpu
from jax.experimental.pallas import tpu_sc as plsc

N = 10000
E = 320000
NC, NS, L = 2, 16, 16
NW = NC * NS
CHUNK = 128
CH = 80
EPW = CH * CHUNK
EPAD = NW * EPW
NPAD = 10240
RPW = NPAD // NS
D1 = 128
D2 = 48
BR = 1000
GRID = N // BR


def _mesh():
    return plsc.VectorSubcoreMesh(
        core_axis_name="c", subcore_axis_name="s", num_cores=NC, num_subcores=NS
    )


_SC_PARAMS = pltpu.CompilerParams(use_tc_tiling_on_sc=False)


def _deg_body(dst_hbm, w1_hbm, w2_hbm, out_hbm, dst_v, w1_v, w2_v, zb, acc1, acc2):
    c = lax.axis_index("c")
    s = lax.axis_index("s")
    wid = c * NS + s

    def zb_init(i, _):
        zb[pl.ds(i * L, L)] = jnp.zeros((L,), jnp.float32)
        return 0

    lax.fori_loop(0, RPW // L, zb_init, 0)
    pltpu.sync_copy(zb, acc1.at[pl.ds(s * RPW, RPW)])
    pltpu.sync_copy(zb, acc2.at[pl.ds(s * RPW, RPW)])
    plsc.subcore_barrier()

    pltpu.sync_copy(dst_hbm.at[wid], dst_v)
    pltpu.sync_copy(w1_hbm.at[wid], w1_v)
    pltpu.sync_copy(w2_hbm.at[wid], w2_v)

    def step(j, _):
        pltpu.sync_copy(w1_v.at[j], acc1.at[dst_v.at[j]], add=True)
        pltpu.sync_copy(w2_v.at[j], acc2.at[dst_v.at[j]], add=True)
        return 0

    lax.fori_loop(0, CH, step, 0)
    plsc.subcore_barrier()
    pltpu.sync_copy(acc1.at[pl.ds(s * RPW, RPW)], out_hbm.at[c, 0, pl.ds(s * RPW, RPW)])
    pltpu.sync_copy(acc2.at[pl.ds(s * RPW, RPW)], out_hbm.at[c, 1, pl.ds(s * RPW, RPW)])


def _deg_call(dstp, w1p, w2p):
    f = pl.kernel(
        _deg_body,
        out_type=jax.ShapeDtypeStruct((NC, 2, NPAD), jnp.float32),
        mesh=_mesh(),
        compiler_params=_SC_PARAMS,
        scratch_types=[
            pltpu.VMEM((CH, CHUNK), jnp.int32),
            pltpu.VMEM((CH, CHUNK), jnp.float32),
            pltpu.VMEM((CH, CHUNK), jnp.float32),
            pltpu.VMEM((RPW,), jnp.float32),
            pltpu.VMEM_SHARED((NPAD,), jnp.float32),
            pltpu.VMEM_SHARED((NPAD,), jnp.float32),
        ],
    )
    return f(dstp, w1p, w2p)


DH = 64


def _zero_acc(buf, acc, s, D):
    nq = D // L

    def zrow(r, _):
        for q in range(nq):
            buf[r, pl.ds(q * L, L)] = jnp.zeros((L,), jnp.float32)
        return 0

    lax.fori_loop(0, CHUNK, zrow, 0)
    for k in range(RPW // CHUNK):
        pltpu.sync_copy(buf, acc.at[pl.ds(s * RPW + k * CHUNK, CHUNK)])


def _edge1_body(ha_hbm, hb_hbm, src_hbm, dst_hbm, w_hbm, out_hbm,
                src_v, dst_v, w_v, buf, acc, gsem, wsem, ssem):
    c = lax.axis_index("c")
    s = lax.axis_index("s")
    wid = c * NS + s

    pltpu.sync_copy(src_hbm.at[wid], src_v)
    pltpu.sync_copy(dst_hbm.at[wid], dst_v)

    for phase, h_hbm in enumerate((ha_hbm, hb_hbm)):
        _zero_acc(buf.at[0], acc, s, DH)
        plsc.subcore_barrier()

        pltpu.async_copy(h_hbm.at[src_v.at[0]], buf.at[0], gsem)
        pltpu.async_copy(w_hbm.at[wid, 0], w_v.at[0], wsem)

        def j2step(j2, _):
            for b in range(2):
                j = j2 * 2 + b
                bufb = buf.at[b]
                w_vb = w_v.at[b]
                pltpu.make_async_copy(h_hbm.at[src_v.at[j]], bufb, gsem).wait()
                pltpu.make_async_copy(w_hbm.at[wid, j], w_vb, wsem).wait()

                @pl.when(j >= 1)
                def _():
                    pltpu.make_async_copy(
                        buf.at[1 - b], acc.at[dst_v.at[j - 1]], ssem
                    ).wait()

                @pl.when(j < CH - 1)
                def _():
                    pltpu.async_copy(h_hbm.at[src_v.at[j + 1]], buf.at[1 - b], gsem)
                    pltpu.async_copy(w_hbm.at[wid, j + 1], w_v.at[1 - b], wsem)

                def srow(r4, _):
                    for rr in range(4):
                        r = r4 * 4 + rr
                        w = w_vb[r]
                        for q in range(DH // L):
                            bufb[r, pl.ds(q * L, L)] = bufb[r, pl.ds(q * L, L)] * w
                    return 0

                lax.fori_loop(0, CHUNK // 4, srow, 0)
                pltpu.async_copy(bufb, acc.at[dst_v.at[j]], ssem, add=True)
            return 0

        lax.fori_loop(0, CH // 2, j2step, 0)
        pltpu.make_async_copy(buf.at[1], acc.at[dst_v.at[CH - 1]], ssem).wait()
        plsc.subcore_barrier()
        pltpu.sync_copy(
            acc.at[pl.ds(s * RPW, RPW)], out_hbm.at[c, phase, pl.ds(s * RPW, RPW)]
        )


def _edge1_call(ha, hb, srcp, dstp, wp):
    f = pl.kernel(
        _edge1_body,
        out_type=jax.ShapeDtypeStruct((NC, 2, NPAD, DH), jnp.float32),
        mesh=_mesh(),
        compiler_params=_SC_PARAMS,
        scratch_types=[
            pltpu.VMEM((CH, CHUNK), jnp.int32),
            pltpu.VMEM((CH, CHUNK), jnp.int32),
            pltpu.VMEM((2, CHUNK, L), jnp.float32),
            pltpu.VMEM((2, CHUNK, DH), jnp.float32),
            pltpu.VMEM_SHARED((NPAD, DH), jnp.float32),
            pltpu.SemaphoreType.DMA,
            pltpu.SemaphoreType.DMA,
            pltpu.SemaphoreType.DMA,
        ],
    )
    wrep = jnp.broadcast_to(wp[..., None], (NW, CH, CHUNK, L))
    return f(ha, hb, srcp, dstp, wrep)


def _edge2_body(h_hbm, src_hbm, dst_hbm, out_hbm, src_v, dst_v, buf, acc, gsem, ssem):
    c = lax.axis_index("c")
    s = lax.axis_index("s")
    wid = c * NS + s

    _zero_acc(buf.at[0], acc, s, D2)
    plsc.subcore_barrier()

    pltpu.sync_copy(src_hbm.at[wid], src_v)
    pltpu.sync_copy(dst_hbm.at[wid], dst_v)
    pltpu.async_copy(h_hbm.at[src_v.at[0]], buf.at[0], gsem)

    def j2step(j2, _):
        for b in range(2):
            j = j2 * 2 + b
            bufb = buf.at[b]
            pltpu.make_async_copy(h_hbm.at[src_v.at[j]], bufb, gsem).wait()

            @pl.when(j >= 1)
            def _():
                pltpu.make_async_copy(
                    buf.at[1 - b], acc.at[dst_v.at[j - 1]], ssem
                ).wait()

            @pl.when(j < CH - 1)
            def _():
                pltpu.async_copy(h_hbm.at[src_v.at[j + 1]], buf.at[1 - b], gsem)

            pltpu.async_copy(bufb, acc.at[dst_v.at[j]], ssem, add=True)
        return 0

    lax.fori_loop(0, CH // 2, j2step, 0)
    pltpu.make_async_copy(buf.at[1], acc.at[dst_v.at[CH - 1]], ssem).wait()
    plsc.subcore_barrier()
    pltpu.sync_copy(acc.at[pl.ds(s * RPW, RPW)], out_hbm.at[c, pl.ds(s * RPW, RPW)])


def _edge2_call(h, srcp, dstp):
    f = pl.kernel(
        _edge2_body,
        out_type=jax.ShapeDtypeStruct((NC, NPAD, D2), jnp.float32),
        mesh=_mesh(),
        compiler_params=_SC_PARAMS,
        scratch_types=[
            pltpu.VMEM((CH, CHUNK), jnp.int32),
            pltpu.VMEM((CH, CHUNK), jnp.int32),
            pltpu.VMEM((2, CHUNK, D2), jnp.float32),
            pltpu.VMEM_SHARED((NPAD, D2), jnp.float32),
            pltpu.SemaphoreType.DMA,
            pltpu.SemaphoreType.DMA,
        ],
    )
    return f(h, srcp, dstp)


def _prep_body(x_ref, w1_ref, d1_ref, d2_ref, ha_ref, hb_ref, i1_ref, i2_ref):
    deg1 = d1_ref[0] + d1_ref[1] + 1.0
    deg2 = d2_ref[0] + d2_ref[1] + 1.0
    dinv1 = jnp.where(deg1 > 0, lax.rsqrt(deg1), 0.0)
    dinv2 = jnp.where(deg2 > 0, lax.rsqrt(deg2), 0.0)
    h1 = jnp.dot(x_ref[...], w1_ref[...], preferred_element_type=jnp.float32)
    h1p = h1 * dinv1
    ha_ref[...] = h1p[:, :DH]
    hb_ref[...] = h1p[:, DH:]
    i1_ref[...] = dinv1
    i2_ref[...] = dinv2


def _prep_call(x, W1, d1, d2):
    return pl.pallas_call(
        _prep_body,
        grid=(GRID,),
        in_specs=[
            pl.BlockSpec((BR, D1), lambda i: (i, 0)),
            pl.BlockSpec((D1, D1), lambda i: (0, 0)),
            pl.BlockSpec((NC, BR, 1), lambda i: (0, i, 0)),
            pl.BlockSpec((NC, BR, 1), lambda i: (0, i, 0)),
        ],
        out_specs=[
            pl.BlockSpec((BR, DH), lambda i: (i, 0)),
            pl.BlockSpec((BR, DH), lambda i: (i, 0)),
            pl.BlockSpec((BR, 1), lambda i: (i, 0)),
            pl.BlockSpec((BR, 1), lambda i: (i, 0)),
        ],
        out_shape=[
            jax.ShapeDtypeStruct((N, DH), jnp.float32),
            jax.ShapeDtypeStruct((N, DH), jnp.float32),
            jax.ShapeDtypeStruct((N, 1), jnp.float32),
            jax.ShapeDtypeStruct((N, 1), jnp.float32),
        ],
    )(x, W1, d1, d2)


def _mid_body(p_ref, ha_ref, hb_ref, i1_ref, i2_ref, b1_ref, w2_ref, h2p_ref):
    sa = p_ref[0, 0] + p_ref[1, 0] + ha_ref[...]
    sb = p_ref[0, 1] + p_ref[1, 1] + hb_ref[...]
    ssum = jnp.concatenate([sa, sb], axis=1)
    y = jnp.maximum(ssum * i1_ref[...] + b1_ref[...], 0.0)
    h2 = jnp.dot(y, w2_ref[...], preferred_element_type=jnp.float32)
    h2p_ref[...] = h2 * i2_ref[...]


def _mid_call(p1, ha, hb, i1, i2, b1r, W2pad):
    return pl.pallas_call(
        _mid_body,
        grid=(GRID,),
        in_specs=[
            pl.BlockSpec((NC, 2, BR, DH), lambda i: (0, 0, i, 0)),
            pl.BlockSpec((BR, DH), lambda i: (i, 0)),
            pl.BlockSpec((BR, DH), lambda i: (i, 0)),
            pl.BlockSpec((BR, 1), lambda i: (i, 0)),
            pl.BlockSpec((BR, 1), lambda i: (i, 0)),
            pl.BlockSpec((1, D1), lambda i: (0, 0)),
            pl.BlockSpec((D1, D2), lambda i: (0, 0)),
        ],
        out_specs=pl.BlockSpec((BR, D2), lambda i: (i, 0)),
        out_shape=jax.ShapeDtypeStruct((N, D2), jnp.float32),
    )(p1, ha, hb, i1, i2, b1r, W2pad)


def _out_body(p_ref, h2p_ref, i2_ref, b2_ref, o_ref):
    z = (p_ref[0] + p_ref[1] + h2p_ref[...]) * i2_ref[...] + b2_ref[...]
    mask = lax.broadcasted_iota(jnp.int32, z.shape, 1) < 40
    zm = jnp.where(mask, z, -jnp.inf)
    m = jnp.max(zm, axis=1, keepdims=True)
    e = jnp.where(mask, jnp.exp(z - m), 0.0)
    o_ref[...] = z - m - jnp.log(jnp.sum(e, axis=1, keepdims=True))


def _out_call(p2, h2p, i2, b2r):
    return pl.pallas_call(
        _out_body,
        grid=(GRID,),
        in_specs=[
            pl.BlockSpec((NC, BR, D2), lambda i: (0, i, 0)),
            pl.BlockSpec((BR, D2), lambda i: (i, 0)),
            pl.BlockSpec((BR, 1), lambda i: (i, 0)),
            pl.BlockSpec((1, D2), lambda i: (0, 0)),
        ],
        out_specs=pl.BlockSpec((BR, D2), lambda i: (i, 0)),
        out_shape=jax.ShapeDtypeStruct((N, D2), jnp.float32),
    )(p2, h2p, i2, b2r)


def kernel(x, adj_t, edge_weight, W1, b1, W2, b2):
    x = x.astype(jnp.float32)
    src = adj_t[0].astype(jnp.int32)
    dst = adj_t[1].astype(jnp.int32)
    pad = EPAD - E
    srcp = jnp.concatenate([src, jnp.zeros((pad,), jnp.int32)]).reshape(NW, CH, CHUNK)
    dstp = jnp.concatenate([dst, jnp.full((pad,), N, jnp.int32)]).reshape(NW, CH, CHUNK)
    w1p = jnp.concatenate(
        [edge_weight.astype(jnp.float32), jnp.zeros((pad,), jnp.float32)]
    ).reshape(NW, CH, CHUNK)
    w2p = jnp.concatenate(
        [jnp.ones((E,), jnp.float32), jnp.zeros((pad,), jnp.float32)]
    ).reshape(NW, CH, CHUNK)
    W2pad = jnp.pad(W2, ((0, 0), (0, D2 - 40)))
    b2r = jnp.pad(b2, (0, D2 - 40)).reshape(1, D2)
    b1r = b1.reshape(1, D1)

    degp = _deg_call(dstp, w1p, w2p)
    d1 = degp[:, 0, :N].reshape(NC, N, 1)
    d2 = degp[:, 1, :N].reshape(NC, N, 1)
    ha, hb, i1, i2 = _prep_call(x, W1, d1, d2)
    p1 = _edge1_call(ha, hb, srcp, dstp, w1p)
    h2p = _mid_call(p1, ha, hb, i1, i2, b1r, W2pad)
    p2 = _edge2_call(h2p, srcp, dstp)
    o = _out_call(p2, h2p, i2, b2r)
    return o[:, :40]


# speedup vs baseline: 15.0107x; 1.0793x over previous
import jax
import jax.numpy as jnp
from jax import lax
from jax.experimental import pallas as pl
from jax.experimental.pallas import tpu as pltpu
from jax.experimental.pallas import tpu_sc as plsc

N = 10000
E = 320000
NC, NS, L = 2, 16, 16
NW = NC * NS
CHUNK = 128
CH = 80
EPW = CH * CHUNK
EPAD = NW * EPW
NPAD = 10240
RPW = NPAD // NS
D1 = 128
D2 = 48
DH = 64
NB = 4
BR = 1000
GRID = N // BR


def _mesh():
    return plsc.VectorSubcoreMesh(
        core_axis_name="c", subcore_axis_name="s", num_cores=NC, num_subcores=NS
    )


_SC_PARAMS = pltpu.CompilerParams(use_tc_tiling_on_sc=False)


def _deg_body(dst_hbm, w1_hbm, w2_hbm, out_hbm, dst_v, w1_v, w2_v, zb, acc1, acc2):
    c = lax.axis_index("c")
    s = lax.axis_index("s")
    wid = c * NS + s

    def zb_init(i, _):
        zb[pl.ds(i * L, L)] = jnp.zeros((L,), jnp.float32)
        return 0

    lax.fori_loop(0, RPW // L, zb_init, 0)
    pltpu.sync_copy(zb, acc1.at[pl.ds(s * RPW, RPW)])
    pltpu.sync_copy(zb, acc2.at[pl.ds(s * RPW, RPW)])
    plsc.subcore_barrier()

    pltpu.sync_copy(dst_hbm.at[wid], dst_v)
    pltpu.sync_copy(w1_hbm.at[wid], w1_v)
    pltpu.sync_copy(w2_hbm.at[wid], w2_v)

    def step(j, _):
        pltpu.sync_copy(w1_v.at[j], acc1.at[dst_v.at[j]], add=True)
        pltpu.sync_copy(w2_v.at[j], acc2.at[dst_v.at[j]], add=True)
        return 0

    lax.fori_loop(0, CH, step, 0)
    plsc.subcore_barrier()
    pltpu.sync_copy(acc1.at[pl.ds(s * RPW, RPW)], out_hbm.at[c, 0, pl.ds(s * RPW, RPW)])
    pltpu.sync_copy(acc2.at[pl.ds(s * RPW, RPW)], out_hbm.at[c, 1, pl.ds(s * RPW, RPW)])


def _deg_call(dstp, w1p, w2p):
    f = pl.kernel(
        _deg_body,
        out_type=jax.ShapeDtypeStruct((NC, 2, NPAD), jnp.float32),
        mesh=_mesh(),
        compiler_params=_SC_PARAMS,
        scratch_types=[
            pltpu.VMEM((CH, CHUNK), jnp.int32),
            pltpu.VMEM((CH, CHUNK), jnp.float32),
            pltpu.VMEM((CH, CHUNK), jnp.float32),
            pltpu.VMEM((RPW,), jnp.float32),
            pltpu.VMEM_SHARED((NPAD,), jnp.float32),
            pltpu.VMEM_SHARED((NPAD,), jnp.float32),
        ],
    )
    return f(dstp, w1p, w2p)


def _zero_acc(buf, acc, s, D):
    nq = D // L

    def zrow(r, _):
        for q in range(nq):
            buf[r, pl.ds(q * L, L)] = jnp.zeros((L,), jnp.float32)
        return 0

    lax.fori_loop(0, CHUNK, zrow, 0)
    for k in range(RPW // CHUNK):
        pltpu.sync_copy(buf, acc.at[pl.ds(s * RPW + k * CHUNK, CHUNK)])


def _edge1_body(ha_hbm, hb_hbm, src_hbm, dst_hbm, w_hbm, out_hbm,
                src_v, dst_v, w_v, buf, acc, gsem, wsem, ssem):
    c = lax.axis_index("c")
    s = lax.axis_index("s")
    wid = c * NS + s

    pltpu.sync_copy(src_hbm.at[wid], src_v)
    pltpu.sync_copy(dst_hbm.at[wid], dst_v)

    for phase, h_hbm in enumerate((ha_hbm, hb_hbm)):
        _zero_acc(buf.at[0], acc, s, DH)
        plsc.subcore_barrier()

        for p in range(2):
            pltpu.async_copy(h_hbm.at[src_v.at[p]], buf.at[p], gsem)
            pltpu.async_copy(w_hbm.at[wid, p], w_v.at[p], wsem)

        def jstep(jq, _):
            for b in range(NB):
                j = jq * NB + b
                bufb = buf.at[b]
                w_vb = w_v.at[b]
                pltpu.make_async_copy(h_hbm.at[src_v.at[j]], bufb, gsem).wait()
                pltpu.make_async_copy(w_hbm.at[wid, j], w_vb, wsem).wait()

                @pl.when(j >= 2)
                def _():
                    pltpu.make_async_copy(
                        buf.at[(b + 2) % NB], acc.at[dst_v.at[j - 2]], ssem
                    ).wait()

                @pl.when(j < CH - 2)
                def _():
                    pltpu.async_copy(
                        h_hbm.at[src_v.at[j + 2]], buf.at[(b + 2) % NB], gsem
                    )
                    pltpu.async_copy(
                        w_hbm.at[wid, j + 2], w_v.at[(b + 2) % NB], wsem
                    )

                def srow(r4, _):
                    for rr in range(4):
                        r = r4 * 4 + rr
                        w = w_vb[r]
                        for q in range(DH // L):
                            bufb[r, pl.ds(q * L, L)] = bufb[r, pl.ds(q * L, L)] * w
                    return 0

                lax.fori_loop(0, CHUNK // 4, srow, 0)
                pltpu.async_copy(bufb, acc.at[dst_v.at[j]], ssem, add=True)
            return 0

        lax.fori_loop(0, CH // NB, jstep, 0)
        for p in (CH - 2, CH - 1):
            pltpu.make_async_copy(
                buf.at[p % NB], acc.at[dst_v.at[p]], ssem
            ).wait()
        plsc.subcore_barrier()
        pltpu.sync_copy(
            acc.at[pl.ds(s * RPW, RPW)], out_hbm.at[c, phase, pl.ds(s * RPW, RPW)]
        )


def _edge1_call(ha, hb, srcp, dstp, wp):
    f = pl.kernel(
        _edge1_body,
        out_type=jax.ShapeDtypeStruct((NC, 2, NPAD, DH), jnp.float32),
        mesh=_mesh(),
        compiler_params=_SC_PARAMS,
        scratch_types=[
            pltpu.VMEM((CH, CHUNK), jnp.int32),
            pltpu.VMEM((CH, CHUNK), jnp.int32),
            pltpu.VMEM((NB, CHUNK, L), jnp.float32),
            pltpu.VMEM((NB, CHUNK, DH), jnp.float32),
            pltpu.VMEM_SHARED((NPAD, DH), jnp.float32),
            pltpu.SemaphoreType.DMA,
            pltpu.SemaphoreType.DMA,
            pltpu.SemaphoreType.DMA,
        ],
    )
    wrep = jnp.broadcast_to(wp[..., None], (NW, CH, CHUNK, L))
    return f(ha, hb, srcp, dstp, wrep)


def _edge2_body(h_hbm, src_hbm, dst_hbm, out_hbm, src_v, dst_v, buf, acc, gsem, ssem):
    c = lax.axis_index("c")
    s = lax.axis_index("s")
    wid = c * NS + s

    _zero_acc(buf.at[0], acc, s, D2)
    pltpu.sync_copy(src_hbm.at[wid], src_v)
    pltpu.sync_copy(dst_hbm.at[wid], dst_v)
    plsc.subcore_barrier()

    for p in range(2):
        pltpu.async_copy(h_hbm.at[src_v.at[p]], buf.at[p], gsem)

    def jstep(jq, _):
        for b in range(NB):
            j = jq * NB + b
            bufb = buf.at[b]
            pltpu.make_async_copy(h_hbm.at[src_v.at[j]], bufb, gsem).wait()

            @pl.when(j >= 2)
            def _():
                pltpu.make_async_copy(
                    buf.at[(b + 2) % NB], acc.at[dst_v.at[j - 2]], ssem
                ).wait()

            @pl.when(j < CH - 2)
            def _():
                pltpu.async_copy(
                    h_hbm.at[src_v.at[j + 2]], buf.at[(b + 2) % NB], gsem
                )

            pltpu.async_copy(bufb, acc.at[dst_v.at[j]], ssem, add=True)
        return 0

    lax.fori_loop(0, CH // NB, jstep, 0)
    for p in (CH - 2, CH - 1):
        pltpu.make_async_copy(buf.at[p % NB], acc.at[dst_v.at[p]], ssem).wait()
    plsc.subcore_barrier()
    pltpu.sync_copy(acc.at[pl.ds(s * RPW, RPW)], out_hbm.at[c, pl.ds(s * RPW, RPW)])


def _edge2_call(h, srcp, dstp):
    f = pl.kernel(
        _edge2_body,
        out_type=jax.ShapeDtypeStruct((NC, NPAD, D2), jnp.float32),
        mesh=_mesh(),
        compiler_params=_SC_PARAMS,
        scratch_types=[
            pltpu.VMEM((CH, CHUNK), jnp.int32),
            pltpu.VMEM((CH, CHUNK), jnp.int32),
            pltpu.VMEM((NB, CHUNK, D2), jnp.float32),
            pltpu.VMEM_SHARED((NPAD, D2), jnp.float32),
            pltpu.SemaphoreType.DMA,
            pltpu.SemaphoreType.DMA,
        ],
    )
    return f(h, srcp, dstp)


def _prep_body(x_ref, w1_ref, d1_ref, d2_ref, ha_ref, hb_ref, i1_ref, i2_ref):
    deg1 = d1_ref[0] + d1_ref[1] + 1.0
    deg2 = d2_ref[0] + d2_ref[1] + 1.0
    dinv1 = jnp.where(deg1 > 0, lax.rsqrt(deg1), 0.0)
    dinv2 = jnp.where(deg2 > 0, lax.rsqrt(deg2), 0.0)
    h1 = jnp.dot(x_ref[...], w1_ref[...], preferred_element_type=jnp.float32)
    h1p = h1 * dinv1
    ha_ref[...] = h1p[:, :DH]
    hb_ref[...] = h1p[:, DH:]
    i1_ref[...] = dinv1
    i2_ref[...] = dinv2


def _prep_call(x, W1, d1, d2):
    return pl.pallas_call(
        _prep_body,
        grid=(GRID,),
        in_specs=[
            pl.BlockSpec((BR, D1), lambda i: (i, 0)),
            pl.BlockSpec((D1, D1), lambda i: (0, 0)),
            pl.BlockSpec((NC, BR, 1), lambda i: (0, i, 0)),
            pl.BlockSpec((NC, BR, 1), lambda i: (0, i, 0)),
        ],
        out_specs=[
            pl.BlockSpec((BR, DH), lambda i: (i, 0)),
            pl.BlockSpec((BR, DH), lambda i: (i, 0)),
            pl.BlockSpec((BR, 1), lambda i: (i, 0)),
            pl.BlockSpec((BR, 1), lambda i: (i, 0)),
        ],
        out_shape=[
            jax.ShapeDtypeStruct((N, DH), jnp.float32),
            jax.ShapeDtypeStruct((N, DH), jnp.float32),
            jax.ShapeDtypeStruct((N, 1), jnp.float32),
            jax.ShapeDtypeStruct((N, 1), jnp.float32),
        ],
    )(x, W1, d1, d2)


def _mid_body(p_ref, ha_ref, hb_ref, i1_ref, i2_ref, b1_ref, w2_ref, h2p_ref):
    sa = p_ref[0, 0] + p_ref[1, 0] + ha_ref[...]
    sb = p_ref[0, 1] + p_ref[1, 1] + hb_ref[...]
    ssum = jnp.concatenate([sa, sb], axis=1)
    y = jnp.maximum(ssum * i1_ref[...] + b1_ref[...], 0.0)
    h2 = jnp.dot(y, w2_ref[...], preferred_element_type=jnp.float32)
    h2p_ref[...] = h2 * i2_ref[...]


def _mid_call(p1, ha, hb, i1, i2, b1r, W2):
    return pl.pallas_call(
        _mid_body,
        grid=(GRID,),
        in_specs=[
            pl.BlockSpec((NC, 2, BR, DH), lambda i: (0, 0, i, 0)),
            pl.BlockSpec((BR, DH), lambda i: (i, 0)),
            pl.BlockSpec((BR, DH), lambda i: (i, 0)),
            pl.BlockSpec((BR, 1), lambda i: (i, 0)),
            pl.BlockSpec((BR, 1), lambda i: (i, 0)),
            pl.BlockSpec((1, D1), lambda i: (0, 0)),
            pl.BlockSpec((D1, D2), lambda i: (0, 0)),
        ],
        out_specs=pl.BlockSpec((BR, D2), lambda i: (i, 0)),
        out_shape=jax.ShapeDtypeStruct((N, D2), jnp.float32),
    )(p1, ha, hb, i1, i2, b1r, W2)


def _out_body(p_ref, h2p_ref, i2_ref, b2_ref, o_ref):
    z = (p_ref[0] + p_ref[1] + h2p_ref[...]) * i2_ref[...] + b2_ref[...]
    mask = lax.broadcasted_iota(jnp.int32, z.shape, 1) < 40
    zm = jnp.where(mask, z, -jnp.inf)
    m = jnp.max(zm, axis=1, keepdims=True)
    e = jnp.where(mask, jnp.exp(z - m), 0.0)
    o_ref[...] = z - m - jnp.log(jnp.sum(e, axis=1, keepdims=True))


def _out_call(p2, h2p, i2, b2r):
    return pl.pallas_call(
        _out_body,
        grid=(GRID,),
        in_specs=[
            pl.BlockSpec((NC, BR, D2), lambda i: (0, i, 0)),
            pl.BlockSpec((BR, D2), lambda i: (i, 0)),
            pl.BlockSpec((BR, 1), lambda i: (i, 0)),
            pl.BlockSpec((1, D2), lambda i: (0, 0)),
        ],
        out_specs=pl.BlockSpec((BR, D2), lambda i: (i, 0)),
        out_shape=jax.ShapeDtypeStruct((N, D2), jnp.float32),
    )(p2, h2p, i2, b2r)


def kernel(x, adj_t, edge_weight, W1, b1, W2, b2):
    x = x.astype(jnp.float32)
    src = adj_t[0].astype(jnp.int32)
    dst = adj_t[1].astype(jnp.int32)
    pad = EPAD - E
    srcp = jnp.concatenate([src, jnp.zeros((pad,), jnp.int32)]).reshape(NW, CH, CHUNK)
    dstp = jnp.concatenate([dst, jnp.full((pad,), N, jnp.int32)]).reshape(NW, CH, CHUNK)
    w1p = jnp.concatenate(
        [edge_weight.astype(jnp.float32), jnp.zeros((pad,), jnp.float32)]
    ).reshape(NW, CH, CHUNK)
    w2p = jnp.concatenate(
        [jnp.ones((E,), jnp.float32), jnp.zeros((pad,), jnp.float32)]
    ).reshape(NW, CH, CHUNK)
    b1r = b1.reshape(1, D1)
    W2 = jnp.pad(W2, ((0, 0), (0, D2 - 40)))
    b2r = jnp.pad(b2, (0, D2 - 40)).reshape(1, D2)

    degp = _deg_call(dstp, w1p, w2p)
    d1 = degp[:, 0, :N].reshape(NC, N, 1)
    d2 = degp[:, 1, :N].reshape(NC, N, 1)
    ha, hb, i1, i2 = _prep_call(x, W1, d1, d2)
    p1 = _edge1_call(ha, hb, srcp, dstp, w1p)
    h2p = _mid_call(p1, ha, hb, i1, i2, b1r, W2)
    p2 = _edge2_call(h2p, srcp, dstp)
    return _out_call(p2, h2p, i2, b2r)[:, :40]


# speedup vs baseline: 15.0119x; 1.0001x over previous
import jax
import jax.numpy as jnp
from jax import lax
from jax.experimental import pallas as pl
from jax.experimental.pallas import tpu as pltpu
from jax.experimental.pallas import tpu_sc as plsc

N = 10000
E = 320000
NC, NS, L = 2, 16, 16
NW = NC * NS
CHUNK = 128
CH = 80
EPW = CH * CHUNK
EPAD = NW * EPW
NPAD = 10240
RPW = NPAD // NS
D1 = 128
D2 = 48
DH = 64
NB = 4
BR = 1000
GRID = N // BR


def _mesh():
    return plsc.VectorSubcoreMesh(
        core_axis_name="c", subcore_axis_name="s", num_cores=NC, num_subcores=NS
    )


_SC_PARAMS = pltpu.CompilerParams(use_tc_tiling_on_sc=False)


def _deg_body(dst_hbm, w1_hbm, w2_hbm, out_hbm, dst_v, w1_v, w2_v, zb, acc1, acc2):
    c = lax.axis_index("c")
    s = lax.axis_index("s")
    wid = c * NS + s

    def zb_init(i, _):
        zb[pl.ds(i * L, L)] = jnp.zeros((L,), jnp.float32)
        return 0

    lax.fori_loop(0, RPW // L, zb_init, 0)
    pltpu.sync_copy(zb, acc1.at[pl.ds(s * RPW, RPW)])
    pltpu.sync_copy(zb, acc2.at[pl.ds(s * RPW, RPW)])
    plsc.subcore_barrier()

    pltpu.sync_copy(dst_hbm.at[wid], dst_v)
    pltpu.sync_copy(w1_hbm.at[wid], w1_v)
    pltpu.sync_copy(w2_hbm.at[wid], w2_v)

    def step(j, _):
        pltpu.sync_copy(w1_v.at[j], acc1.at[dst_v.at[j]], add=True)
        pltpu.sync_copy(w2_v.at[j], acc2.at[dst_v.at[j]], add=True)
        return 0

    lax.fori_loop(0, CH, step, 0)
    plsc.subcore_barrier()
    pltpu.sync_copy(acc1.at[pl.ds(s * RPW, RPW)], out_hbm.at[c, 0, pl.ds(s * RPW, RPW)])
    pltpu.sync_copy(acc2.at[pl.ds(s * RPW, RPW)], out_hbm.at[c, 1, pl.ds(s * RPW, RPW)])


def _deg_call(dstp, w1p, w2p):
    f = pl.kernel(
        _deg_body,
        out_type=jax.ShapeDtypeStruct((NC, 2, NPAD), jnp.float32),
        mesh=_mesh(),
        compiler_params=_SC_PARAMS,
        scratch_types=[
            pltpu.VMEM((CH, CHUNK), jnp.int32),
            pltpu.VMEM((CH, CHUNK), jnp.float32),
            pltpu.VMEM((CH, CHUNK), jnp.float32),
            pltpu.VMEM((RPW,), jnp.float32),
            pltpu.VMEM_SHARED((NPAD,), jnp.float32),
            pltpu.VMEM_SHARED((NPAD,), jnp.float32),
        ],
    )
    return f(dstp, w1p, w2p)


def _zero_acc(buf, acc, s, D):
    nq = D // L

    def zrow(r, _):
        for q in range(nq):
            buf[r, pl.ds(q * L, L)] = jnp.zeros((L,), jnp.float32)
        return 0

    lax.fori_loop(0, CHUNK, zrow, 0)
    for k in range(RPW // CHUNK):
        pltpu.sync_copy(buf, acc.at[pl.ds(s * RPW + k * CHUNK, CHUNK)])


def _edge1_body(ha_hbm, hb_hbm, src_hbm, dst_hbm, w_hbm, out_hbm,
                src_v, dst_v, w_v, buf, acc, gsem, wsem, ssem):
    c = lax.axis_index("c")
    s = lax.axis_index("s")
    wid = c * NS + s

    pltpu.sync_copy(src_hbm.at[wid], src_v)
    pltpu.sync_copy(dst_hbm.at[wid], dst_v)

    for phase, h_hbm in enumerate((ha_hbm, hb_hbm)):
        _zero_acc(buf.at[0], acc, s, DH)
        plsc.subcore_barrier()

        for p in range(2):
            pltpu.async_copy(h_hbm.at[src_v.at[p]], buf.at[p], gsem)
            pltpu.async_copy(w_hbm.at[wid, p], w_v.at[p], wsem)

        def jstep(jq, _):
            for b in range(NB):
                j = jq * NB + b
                bufb = buf.at[b]
                w_vb = w_v.at[b]
                pltpu.make_async_copy(h_hbm.at[src_v.at[j]], bufb, gsem).wait()
                pltpu.make_async_copy(w_hbm.at[wid, j], w_vb, wsem).wait()

                @pl.when(j >= 2)
                def _():
                    pltpu.make_async_copy(
                        buf.at[(b + 2) % NB], acc.at[dst_v.at[j - 2]], ssem
                    ).wait()

                @pl.when(j < CH - 2)
                def _():
                    pltpu.async_copy(
                        h_hbm.at[src_v.at[j + 2]], buf.at[(b + 2) % NB], gsem
                    )
                    pltpu.async_copy(
                        w_hbm.at[wid, j + 2], w_v.at[(b + 2) % NB], wsem
                    )

                def srow(r4, _):
                    for rr in range(4):
                        r = r4 * 4 + rr
                        w = w_vb[r]
                        for q in range(DH // L):
                            bufb[r, pl.ds(q * L, L)] = bufb[r, pl.ds(q * L, L)] * w
                    return 0

                lax.fori_loop(0, CHUNK // 4, srow, 0)
                pltpu.async_copy(bufb, acc.at[dst_v.at[j]], ssem, add=True)
            return 0

        lax.fori_loop(0, CH // NB, jstep, 0)
        for p in (CH - 2, CH - 1):
            pltpu.make_async_copy(
                buf.at[p % NB], acc.at[dst_v.at[p]], ssem
            ).wait()
        plsc.subcore_barrier()
        pltpu.sync_copy(
            acc.at[pl.ds(s * RPW, RPW)], out_hbm.at[c, phase, pl.ds(s * RPW, RPW)]
        )


def _edge1_call(ha, hb, srcp, dstp, wp):
    f = pl.kernel(
        _edge1_body,
        out_type=jax.ShapeDtypeStruct((NC, 2, NPAD, DH), jnp.float32),
        mesh=_mesh(),
        compiler_params=_SC_PARAMS,
        scratch_types=[
            pltpu.VMEM((CH, CHUNK), jnp.int32),
            pltpu.VMEM((CH, CHUNK), jnp.int32),
            pltpu.VMEM((NB, CHUNK, L), jnp.float32),
            pltpu.VMEM((NB, CHUNK, DH), jnp.float32),
            pltpu.VMEM_SHARED((NPAD, DH), jnp.float32),
            pltpu.SemaphoreType.DMA,
            pltpu.SemaphoreType.DMA,
            pltpu.SemaphoreType.DMA,
        ],
    )
    wrep = jnp.broadcast_to(wp[..., None], (NW, CH, CHUNK, L))
    return f(ha, hb, srcp, dstp, wrep)


def _edge2_body(h_hbm, src_hbm, dst_hbm, out_hbm, src_v, dst_v, buf, acc, gsem, ssem):
    c = lax.axis_index("c")
    s = lax.axis_index("s")
    wid = c * NS + s

    _zero_acc(buf.at[0], acc, s, D2)
    pltpu.sync_copy(src_hbm.at[wid], src_v)
    pltpu.sync_copy(dst_hbm.at[wid], dst_v)
    plsc.subcore_barrier()

    for p in range(2):
        pltpu.async_copy(h_hbm.at[src_v.at[p]], buf.at[p], gsem)

    def jstep(jq, _):
        for b in range(NB):
            j = jq * NB + b
            bufb = buf.at[b]
            pltpu.make_async_copy(h_hbm.at[src_v.at[j]], bufb, gsem).wait()

            @pl.when(j >= 2)
            def _():
                pltpu.make_async_copy(
                    buf.at[(b + 2) % NB], acc.at[dst_v.at[j - 2]], ssem
                ).wait()

            @pl.when(j < CH - 2)
            def _():
                pltpu.async_copy(
                    h_hbm.at[src_v.at[j + 2]], buf.at[(b + 2) % NB], gsem
                )

            pltpu.async_copy(bufb, acc.at[dst_v.at[j]], ssem, add=True)
        return 0

    lax.fori_loop(0, CH // NB, jstep, 0)
    for p in (CH - 2, CH - 1):
        pltpu.make_async_copy(buf.at[p % NB], acc.at[dst_v.at[p]], ssem).wait()
    plsc.subcore_barrier()
    pltpu.sync_copy(acc.at[pl.ds(s * RPW, RPW)], out_hbm.at[c, pl.ds(s * RPW, RPW)])


def _edge2_call(h, srcp, dstp):
    f = pl.kernel(
        _edge2_body,
        out_type=jax.ShapeDtypeStruct((NC, NPAD, D2), jnp.float32),
        mesh=_mesh(),
        compiler_params=_SC_PARAMS,
        scratch_types=[
            pltpu.VMEM((CH, CHUNK), jnp.int32),
            pltpu.VMEM((CH, CHUNK), jnp.int32),
            pltpu.VMEM((NB, CHUNK, D2), jnp.float32),
            pltpu.VMEM_SHARED((NPAD, D2), jnp.float32),
            pltpu.SemaphoreType.DMA,
            pltpu.SemaphoreType.DMA,
        ],
    )
    return f(h, srcp, dstp)


def _prep_body(x_ref, w1_ref, d1_ref, d2_ref, ha_ref, hb_ref, i1_ref, i2_ref):
    deg1 = d1_ref[0] + d1_ref[1] + 1.0
    deg2 = d2_ref[0] + d2_ref[1] + 1.0
    dinv1 = jnp.where(deg1 > 0, lax.rsqrt(deg1), 0.0)
    dinv2 = jnp.where(deg2 > 0, lax.rsqrt(deg2), 0.0)
    h1 = jnp.dot(x_ref[...], w1_ref[...], preferred_element_type=jnp.float32)
    h1p = h1 * dinv1
    ha_ref[...] = h1p[:, :DH]
    hb_ref[...] = h1p[:, DH:]
    i1_ref[...] = dinv1
    i2_ref[...] = dinv2


def _prep_call(x, W1, d1, d2):
    return pl.pallas_call(
        _prep_body,
        grid=(GRID,),
        in_specs=[
            pl.BlockSpec((BR, D1), lambda i: (i, 0)),
            pl.BlockSpec((D1, D1), lambda i: (0, 0)),
            pl.BlockSpec((NC, BR, 1), lambda i: (0, i, 0)),
            pl.BlockSpec((NC, BR, 1), lambda i: (0, i, 0)),
        ],
        out_specs=[
            pl.BlockSpec((BR, DH), lambda i: (i, 0)),
            pl.BlockSpec((BR, DH), lambda i: (i, 0)),
            pl.BlockSpec((BR, 1), lambda i: (i, 0)),
            pl.BlockSpec((BR, 1), lambda i: (i, 0)),
        ],
        out_shape=[
            jax.ShapeDtypeStruct((N, DH), jnp.float32),
            jax.ShapeDtypeStruct((N, DH), jnp.float32),
            jax.ShapeDtypeStruct((N, 1), jnp.float32),
            jax.ShapeDtypeStruct((N, 1), jnp.float32),
        ],
    )(x, W1, d1, d2)


def _mid_body(p_ref, ha_ref, hb_ref, i1_ref, i2_ref, b1_ref, w2_ref, h2p_ref):
    sa = p_ref[0, 0] + p_ref[1, 0] + ha_ref[...]
    sb = p_ref[0, 1] + p_ref[1, 1] + hb_ref[...]
    ssum = jnp.concatenate([sa, sb], axis=1)
    y = jnp.maximum(ssum * i1_ref[...] + b1_ref[...], 0.0)
    h2 = jnp.dot(y, w2_ref[...], preferred_element_type=jnp.float32)
    h2p_ref[...] = h2 * i2_ref[...]


def _mid_call(p1, ha, hb, i1, i2, b1r, W2):
    return pl.pallas_call(
        _mid_body,
        grid=(GRID,),
        in_specs=[
            pl.BlockSpec((NC, 2, BR, DH), lambda i: (0, 0, i, 0)),
            pl.BlockSpec((BR, DH), lambda i: (i, 0)),
            pl.BlockSpec((BR, DH), lambda i: (i, 0)),
            pl.BlockSpec((BR, 1), lambda i: (i, 0)),
            pl.BlockSpec((BR, 1), lambda i: (i, 0)),
            pl.BlockSpec((1, D1), lambda i: (0, 0)),
            pl.BlockSpec((D1, D2), lambda i: (0, 0)),
        ],
        out_specs=pl.BlockSpec((BR, D2), lambda i: (i, 0)),
        out_shape=jax.ShapeDtypeStruct((N, D2), jnp.float32),
    )(p1, ha, hb, i1, i2, b1r, W2)


def _out_body(p_ref, h2p_ref, i2_ref, b2_ref, o_ref):
    z = (p_ref[0] + p_ref[1] + h2p_ref[...]) * i2_ref[...] + b2_ref[...]
    mask = lax.broadcasted_iota(jnp.int32, z.shape, 1) < 40
    zm = jnp.where(mask, z, -jnp.inf)
    m = jnp.max(zm, axis=1, keepdims=True)
    e = jnp.where(mask, jnp.exp(z - m), 0.0)
    o_ref[...] = z - m - jnp.log(jnp.sum(e, axis=1, keepdims=True))


def _out_call(p2, h2p, i2, b2r):
    return pl.pallas_call(
        _out_body,
        grid=(GRID,),
        in_specs=[
            pl.BlockSpec((NC, BR, D2), lambda i: (0, i, 0)),
            pl.BlockSpec((BR, D2), lambda i: (i, 0)),
            pl.BlockSpec((BR, 1), lambda i: (i, 0)),
            pl.BlockSpec((1, D2), lambda i: (0, 0)),
        ],
        out_specs=pl.BlockSpec((BR, D2), lambda i: (i, 0)),
        out_shape=jax.ShapeDtypeStruct((N, D2), jnp.float32),
    )(p2, h2p, i2, b2r)


def kernel(x, adj_t, edge_weight, W1, b1, W2, b2):
    x = x.astype(jnp.float32)
    src = adj_t[0].astype(jnp.int32)
    dst = adj_t[1].astype(jnp.int32)
    pad = EPAD - E
    srcp = jnp.concatenate([src, jnp.zeros((pad,), jnp.int32)]).reshape(NW, CH, CHUNK)
    pad_dst = N + jnp.arange(pad, dtype=jnp.int32) % (NPAD - N)
    dstp = jnp.concatenate([dst, pad_dst]).reshape(NW, CH, CHUNK)
    w1p = jnp.concatenate(
        [edge_weight.astype(jnp.float32), jnp.zeros((pad,), jnp.float32)]
    ).reshape(NW, CH, CHUNK)
    w2p = jnp.concatenate(
        [jnp.ones((E,), jnp.float32), jnp.zeros((pad,), jnp.float32)]
    ).reshape(NW, CH, CHUNK)
    b1r = b1.reshape(1, D1)
    W2 = jnp.pad(W2, ((0, 0), (0, D2 - 40)))
    b2r = jnp.pad(b2, (0, D2 - 40)).reshape(1, D2)

    degp = _deg_call(dstp, w1p, w2p)
    d1 = degp[:, 0, :N].reshape(NC, N, 1)
    d2 = degp[:, 1, :N].reshape(NC, N, 1)
    ha, hb, i1, i2 = _prep_call(x, W1, d1, d2)
    p1 = _edge1_call(ha, hb, srcp, dstp, w1p)
    h2p = _mid_call(p1, ha, hb, i1, i2, b1r, W2)
    p2 = _edge2_call(h2p, srcp, dstp)
    return _out_call(p2, h2p, i2, b2r)[:, :40]
